# baseline (device time: 2451536 ns/iter reference)
import jax
import jax.numpy as jnp
from jax import lax
from jax.experimental import pallas as pl
from jax.experimental.pallas import tpu as pltpu

E = 8
EL = 4
C = 576
TJ = 512
TM = 384
KWIN = 256


def _row_gather(src, idx):
    n_out = idx.shape[0]
    d = src.shape[1]

    def body(idx_ref, src_ref, out_ref, sem):
        def loop(i, carry):
            @pl.when(i < n_out)
            def _():
                pltpu.make_async_copy(
                    src_ref.at[pl.ds(idx_ref[i], 1)],
                    out_ref.at[pl.ds(i, 1)],
                    sem,
                ).start()

            @pl.when(i >= KWIN)
            def _():
                pltpu.make_async_copy(
                    src_ref.at[pl.ds(0, 1)], out_ref.at[pl.ds(0, 1)], sem
                ).wait()

            return carry

        lax.fori_loop(0, n_out + KWIN, loop, 0)

    return pl.pallas_call(
        body,
        out_shape=jax.ShapeDtypeStruct((n_out, d), src.dtype),
        in_specs=[
            pl.BlockSpec(memory_space=pltpu.SMEM),
            pl.BlockSpec(memory_space=pl.ANY),
        ],
        out_specs=pl.BlockSpec(memory_space=pl.ANY),
        scratch_shapes=[pltpu.SemaphoreType.DMA],
    )(idx, src)


def _row_gather2(a, b, use_b, row):
    n_out = use_b.shape[0]
    d = a.shape[1]

    def body(use_ref, row_ref, a_ref, b_ref, out_ref, sem):
        def loop(i, carry):
            @pl.when(i < n_out)
            def _():
                r = row_ref[i]

                @pl.when(use_ref[i] == 0)
                def _():
                    pltpu.make_async_copy(
                        a_ref.at[pl.ds(r, 1)], out_ref.at[pl.ds(i, 1)], sem
                    ).start()

                @pl.when(use_ref[i] != 0)
                def _():
                    pltpu.make_async_copy(
                        b_ref.at[pl.ds(r, 1)], out_ref.at[pl.ds(i, 1)], sem
                    ).start()

            @pl.when(i >= KWIN)
            def _():
                pltpu.make_async_copy(
                    a_ref.at[pl.ds(0, 1)], out_ref.at[pl.ds(0, 1)], sem
                ).wait()

            return carry

        lax.fori_loop(0, n_out + KWIN, loop, 0)

    return pl.pallas_call(
        body,
        out_shape=jax.ShapeDtypeStruct((n_out, d), a.dtype),
        in_specs=[
            pl.BlockSpec(memory_space=pltpu.SMEM),
            pl.BlockSpec(memory_space=pltpu.SMEM),
            pl.BlockSpec(memory_space=pl.ANY),
            pl.BlockSpec(memory_space=pl.ANY),
        ],
        out_specs=pl.BlockSpec(memory_space=pl.ANY),
        scratch_shapes=[pltpu.SemaphoreType.DMA],
    )(use_b, row, a, b)


def _a2a_exchange(t, cid):

    def body(src_ref, dst_ref, send_sem, recv_sem):
        my_x = lax.axis_index("x")
        my_y = lax.axis_index("y")
        my_z = lax.axis_index("z")
        nbr = (1 - my_x, my_y, my_z)

        barrier_sem = pltpu.get_barrier_semaphore()
        pl.semaphore_signal(
            barrier_sem, inc=1, device_id=nbr,
            device_id_type=pl.DeviceIdType.MESH,
        )
        pl.semaphore_wait(barrier_sem, 1)

        rdma = pltpu.make_async_remote_copy(
            src_ref=src_ref,
            dst_ref=dst_ref,
            send_sem=send_sem,
            recv_sem=recv_sem,
            device_id=nbr,
            device_id_type=pl.DeviceIdType.MESH,
        )
        rdma.start()
        rdma.wait()

    return pl.pallas_call(
        body,
        out_shape=jax.ShapeDtypeStruct(t.shape, t.dtype),
        in_specs=[pl.BlockSpec(memory_space=pl.ANY)],
        out_specs=pl.BlockSpec(memory_space=pl.ANY),
        scratch_shapes=[pltpu.SemaphoreType.DMA, pltpu.SemaphoreType.DMA],
        compiler_params=pltpu.CompilerParams(collective_id=cid),
    )(t)


def _expert_ffn(X, W1, W2):
    n_tok = X.shape[1]
    d_model = X.shape[2]
    d_ff = W1.shape[2]
    J = d_ff // TJ
    M = n_tok // TM

    def body(x_ref, w1_ref, w2_ref, out_ref):
        j = pl.program_id(2)

        @pl.when(j == 0)
        def _():
            out_ref[...] = jnp.zeros_like(out_ref)

        h = jnp.maximum(
            jnp.dot(x_ref[0], w1_ref[0], preferred_element_type=jnp.float32),
            0.0,
        )
        out_ref[0] += jnp.dot(h, w2_ref[0], preferred_element_type=jnp.float32)

    return pl.pallas_call(
        body,
        grid=(EL, M, J),
        in_specs=[
            pl.BlockSpec((1, TM, d_model), lambda e, m, j: (e, m, 0)),
            pl.BlockSpec((1, d_model, TJ), lambda e, m, j: (e, 0, j)),
            pl.BlockSpec((1, TJ, d_model), lambda e, m, j: (e, j, 0)),
        ],
        out_specs=pl.BlockSpec((1, TM, d_model), lambda e, m, j: (e, m, 0)),
        out_shape=jax.ShapeDtypeStruct((EL, n_tok, d_model), jnp.float32),
        compiler_params=pltpu.CompilerParams(
            dimension_semantics=("arbitrary", "arbitrary", "arbitrary"),
        ),
    )(X, W1, W2)


def kernel(x, assign, W1, W2):
    n, d = x.shape
    p = lax.axis_index("x")
    q = 1 - p

    e = assign.astype(jnp.int32)
    onehot = (e[:, None] == jnp.arange(E, dtype=jnp.int32)[None, :]).astype(
        jnp.int32
    )
    slot = jnp.take_along_axis(
        jnp.cumsum(onehot, axis=0), e[:, None], axis=1
    )[:, 0] - 1
    bidx_tok = e * C + slot
    tok_for_slot = (
        jnp.zeros((E * C,), jnp.int32).at[bidx_tok].set(
            jnp.arange(n, dtype=jnp.int32)
        )
    )

    xs = _row_gather(x, tok_for_slot).reshape(E, C, d)

    keep = lax.dynamic_slice(xs, (EL * p, 0, 0), (EL, C, d))
    send = lax.dynamic_slice(xs, (EL * q, 0, 0), (EL, C, d))

    xr = _a2a_exchange(send, cid=0)

    X = jnp.concatenate([keep, xr], axis=1)
    Y = _expert_ffn(X, W1, W2)

    yr = _a2a_exchange(Y[:, C:, :], cid=1)

    local = (e // EL) == p
    row_local = (e - EL * p) * (2 * C) + slot
    row_rem = (e - EL * q) * C + slot
    row = jnp.where(local, row_local, row_rem).astype(jnp.int32)
    use_yr = (~local).astype(jnp.int32)
    return _row_gather2(
        Y.reshape(EL * 2 * C, d), yr.reshape(EL * C, d), use_yr, row
    )
